# baseline (speedup 1.0000x reference)
import jax
import jax.numpy as jnp
from jax import lax
from jax.experimental import pallas as pl
from jax.experimental.pallas import tpu as pltpu
from jax.experimental.pallas import tpu_sc as plsc

N, D, E = 10000, 128, 320000
NC, NS, L = 2, 16, 16
NW = NC * NS
C = 80
CHUNKS = E // (NW * C)
PER_W = CHUNKS * C
NCHG = NW * CHUNKS
N_PAD = 10240
ROWS_PER_TILE = N_PAD // NS
NBUF = 4
NI = 8
OUTER = -(-CHUNKS // NI)


def _sc_body(h_hbm, ei_hbm, e_hbm, out_hbm,
             src8, dst8, ev8, rows4, acc_sh, idx_sem, gat_sem, sct_sem):
    cid = lax.axis_index("c")
    sid = lax.axis_index("s")
    wid = sid * NC + cid

    def zero_body(i, _):
        for f in range(D // L):
            rows4[0, i, pl.ds(f * L, L)] = jnp.zeros((L,), jnp.float32)
        return 0

    lax.fori_loop(0, C, zero_body, 0)
    row0 = sid * ROWS_PER_TILE
    for k in range(ROWS_PER_TILE // C):
        pltpu.async_copy(rows4.at[0], acc_sh.at[pl.ds(row0 + k * C, C)],
                         sct_sem)
    for k in range(ROWS_PER_TILE // C):
        pltpu.make_async_copy(h_hbm.at[pl.ds(0, C)], rows4.at[0],
                              sct_sem).wait()
    plsc.subcore_barrier()

    cbase = wid * CHUNKS

    def issue_idx(k, b):
        base = (cbase + k) * C
        pltpu.async_copy(ei_hbm.at[pl.ds(base, C)], src8.at[b], idx_sem)
        pltpu.async_copy(ei_hbm.at[pl.ds(E + base, C)], dst8.at[b], idx_sem)
        pltpu.async_copy(e_hbm.at[pl.ds(base, C)], ev8.at[b], idx_sem)

    def wait_idx(b):
        pltpu.make_async_copy(ei_hbm.at[pl.ds(0, C)], src8.at[b],
                              idx_sem).wait()
        pltpu.make_async_copy(ei_hbm.at[pl.ds(0, C)], dst8.at[b],
                              idx_sem).wait()
        pltpu.make_async_copy(e_hbm.at[pl.ds(0, C)], ev8.at[b],
                              idx_sem).wait()

    def issue_gat(ib, rb):
        pltpu.async_copy(h_hbm.at[src8.at[ib]], rows4.at[rb], gat_sem)

    def wait_gat(rb):
        pltpu.make_async_copy(h_hbm.at[pl.ds(0, C)], rows4.at[rb],
                              gat_sem).wait()

    def issue_sct(ib, rb):
        pltpu.async_copy(rows4.at[rb], acc_sh.at[dst8.at[ib]], sct_sem,
                         add=True)

    def wait_sct():
        pltpu.make_async_copy(h_hbm.at[pl.ds(0, C)], rows4.at[0],
                              sct_sem).wait()

    def scale(ib, rb):
        def group_body(g, _):
            ev = ev8[ib, pl.ds(g * L, L)]
            for j in range(L):
                i = g * L + j
                eb = ev[j]
                for f in range(D // L):
                    rows4[rb, i, pl.ds(f * L, L)] = (
                        rows4[rb, i, pl.ds(f * L, L)] * eb)
            return 0

        lax.fori_loop(0, C // L, group_body, 0, unroll=C // L)

    for b in range(4):
        issue_idx(b, b)
    wait_idx(0)
    issue_gat(0, 0)
    wait_idx(1)
    issue_gat(1, 1)

    def outer(g, _):
        for j in range(NI):
            k = g * NI + j
            rb = j % NBUF
            ib = j
            ib2 = (j + 2) % NI
            ib4 = (j + 4) % NI
            rb2 = (j + 2) % NBUF

            @pl.when(k <= CHUNKS - 1)
            def _():
                wait_gat(rb)

            @pl.when((k >= 2) & (k <= CHUNKS + 1))
            def _():
                wait_sct()

            @pl.when(k <= CHUNKS - 3)
            def _():
                wait_idx(ib2)
                issue_gat(ib2, rb2)

            @pl.when(k <= CHUNKS - 5)
            def _():
                issue_idx(k + 4, ib4)

            @pl.when(k <= CHUNKS - 1)
            def _():
                scale(ib, rb)
                issue_sct(ib, rb)
        return 0

    lax.fori_loop(0, OUTER, outer, 0)
    plsc.subcore_barrier()

    pltpu.sync_copy(acc_sh.at[pl.ds(row0, ROWS_PER_TILE)],
                    out_hbm.at[cid, pl.ds(row0, ROWS_PER_TILE)])


@jax.jit
def _sc_scatter(h, ei, e_p):
    mesh = plsc.VectorSubcoreMesh(core_axis_name="c", subcore_axis_name="s")
    return pl.kernel(
        _sc_body,
        out_type=jax.ShapeDtypeStruct((NC, N_PAD, D), jnp.float32),
        mesh=mesh,
        scratch_types=[
            pltpu.VMEM((NI, C), jnp.int32),
            pltpu.VMEM((NI, C), jnp.int32),
            pltpu.VMEM((NI, C), jnp.float32),
            pltpu.VMEM((NBUF, C, D), jnp.float32),
            pltpu.VMEM_SHARED((N_PAD, D), jnp.float32),
            pltpu.SemaphoreType.DMA,
            pltpu.SemaphoreType.DMA,
            pltpu.SemaphoreType.DMA,
        ],
    )(h, ei, e_p)


OUT_ROWS = (N * D + E) // D


def _tc_body(alpha_ref, parts_ref, h_ref, out_ref):
    s = jax.nn.sigmoid(alpha_ref[0, 0])
    out_ref[:N] = s * (parts_ref[0, :N] + parts_ref[1, :N] - h_ref[...])
    out_ref[N:] = jnp.zeros((OUT_ROWS - N, D), jnp.float32)


@jax.jit
def _tc_combine(alpha, parts, h):
    return pl.pallas_call(
        _tc_body,
        out_shape=jax.ShapeDtypeStruct((OUT_ROWS, D), jnp.float32),
        in_specs=[
            pl.BlockSpec(memory_space=pltpu.SMEM),
            pl.BlockSpec(memory_space=pltpu.VMEM),
            pl.BlockSpec(memory_space=pltpu.VMEM),
        ],
        out_specs=pl.BlockSpec(memory_space=pltpu.VMEM),
    )(alpha, parts, h)


def kernel(t, x, edge_index, alpha):
    h = x[: N * D].reshape(N, D)
    e = x[N * D:]
    parts = _sc_scatter(h, edge_index.reshape(-1), e)
    out = _tc_combine(jnp.reshape(alpha, (1, 1)), parts, h)
    return out.reshape(-1)


# speedup vs baseline: 1.3931x; 1.3931x over previous
import jax
import jax.numpy as jnp
from jax import lax
from jax.experimental import pallas as pl
from jax.experimental.pallas import tpu as pltpu
from jax.experimental.pallas import tpu_sc as plsc

N, D, E = 10000, 128, 320000
NC, NS, L = 2, 16, 16
NW = NC * NS
C = 80
CHUNKS = E // (NW * C)
PER_W = CHUNKS * C
NCHG = NW * CHUNKS
N_PAD = 10240
ROWS_PER_TILE = N_PAD // NS
NBUF = 4
NI = 8
OUTER = -(-CHUNKS // NI)


def _sc_body(h_hbm, ei_hbm, e_hbm, out_hbm,
             src8, dst8, ev8, rows4, acc_sh, idx_sem, gat_sem, sct_sem):
    cid = lax.axis_index("c")
    sid = lax.axis_index("s")
    wid = sid * NC + cid

    def zero_body(i, _):
        for f in range(D // L):
            rows4[0, i, pl.ds(f * L, L)] = jnp.zeros((L,), jnp.float32)
        return 0

    lax.fori_loop(0, C, zero_body, 0)
    row0 = sid * ROWS_PER_TILE
    for k in range(ROWS_PER_TILE // C):
        pltpu.async_copy(rows4.at[0], acc_sh.at[pl.ds(row0 + k * C, C)],
                         sct_sem)
    for k in range(ROWS_PER_TILE // C):
        pltpu.make_async_copy(h_hbm.at[pl.ds(0, C)], rows4.at[0],
                              sct_sem).wait()
    plsc.subcore_barrier()

    cbase = wid * CHUNKS

    def issue_idx(k, b):
        base = (cbase + k) * C
        pltpu.async_copy(ei_hbm.at[pl.ds(base, C)], src8.at[b], idx_sem)
        pltpu.async_copy(ei_hbm.at[pl.ds(E + base, C)], dst8.at[b], idx_sem)
        pltpu.async_copy(e_hbm.at[pl.ds(base, C)], ev8.at[b], idx_sem)

    def wait_idx(b):
        pltpu.make_async_copy(ei_hbm.at[pl.ds(0, C)], src8.at[b],
                              idx_sem).wait()
        pltpu.make_async_copy(ei_hbm.at[pl.ds(0, C)], dst8.at[b],
                              idx_sem).wait()
        pltpu.make_async_copy(e_hbm.at[pl.ds(0, C)], ev8.at[b],
                              idx_sem).wait()

    def issue_gat(ib, rb):
        pltpu.async_copy(h_hbm.at[src8.at[ib]], rows4.at[rb], gat_sem)

    def wait_gat(rb):
        pltpu.make_async_copy(h_hbm.at[pl.ds(0, C)], rows4.at[rb],
                              gat_sem).wait()

    def issue_sct(ib, rb):
        pltpu.async_copy(rows4.at[rb], acc_sh.at[dst8.at[ib]], sct_sem,
                         add=True)

    def wait_sct():
        pltpu.make_async_copy(h_hbm.at[pl.ds(0, C)], rows4.at[0],
                              sct_sem).wait()

    def scale(ib, rb):
        def group_body(g, _):
            ev = ev8[ib, pl.ds(g * L, L)]
            for j in range(L):
                i = g * L + j
                eb = ev[j]
                for f in range(D // L):
                    rows4[rb, i, pl.ds(f * L, L)] = (
                        rows4[rb, i, pl.ds(f * L, L)] * eb)
            return 0

        lax.fori_loop(0, C // L, group_body, 0)

    for b in range(4):
        issue_idx(b, b)
    wait_idx(0)
    issue_gat(0, 0)
    wait_idx(1)
    issue_gat(1, 1)

    def outer(g, _):
        for j in range(NI):
            k = g * NI + j
            rb = j % NBUF
            ib = j
            ib2 = (j + 2) % NI
            ib4 = (j + 4) % NI
            rb2 = (j + 2) % NBUF

            @pl.when(k <= CHUNKS - 1)
            def _():
                wait_gat(rb)

            @pl.when((k >= 2) & (k <= CHUNKS + 1))
            def _():
                wait_sct()

            @pl.when(k <= CHUNKS - 3)
            def _():
                wait_idx(ib2)
                issue_gat(ib2, rb2)

            @pl.when(k <= CHUNKS - 5)
            def _():
                issue_idx(k + 4, ib4)

            @pl.when(k <= CHUNKS - 1)
            def _():
                scale(ib, rb)
                issue_sct(ib, rb)
        return 0

    lax.fori_loop(0, OUTER, outer, 0)
    plsc.subcore_barrier()

    pltpu.sync_copy(acc_sh.at[pl.ds(row0, ROWS_PER_TILE)],
                    out_hbm.at[cid, pl.ds(row0, ROWS_PER_TILE)])


@jax.jit
def _sc_scatter(h, ei, e_p):
    mesh = plsc.VectorSubcoreMesh(core_axis_name="c", subcore_axis_name="s")
    return pl.kernel(
        _sc_body,
        out_type=jax.ShapeDtypeStruct((NC, N_PAD, D), jnp.float32),
        mesh=mesh,
        scratch_types=[
            pltpu.VMEM((NI, C), jnp.int32),
            pltpu.VMEM((NI, C), jnp.int32),
            pltpu.VMEM((NI, C), jnp.float32),
            pltpu.VMEM((NBUF, C, D), jnp.float32),
            pltpu.VMEM_SHARED((N_PAD, D), jnp.float32),
            pltpu.SemaphoreType.DMA,
            pltpu.SemaphoreType.DMA,
            pltpu.SemaphoreType.DMA,
        ],
    )(h, ei, e_p)


OUT_ROWS = (N * D + E) // D


def _tc_body(alpha_ref, parts_ref, h_ref, out_ref):
    s = jax.nn.sigmoid(alpha_ref[0, 0])
    out_ref[:N] = s * (parts_ref[0, :N] + parts_ref[1, :N] - h_ref[...])
    out_ref[N:] = jnp.zeros((OUT_ROWS - N, D), jnp.float32)


@jax.jit
def _tc_combine(alpha, parts, h):
    return pl.pallas_call(
        _tc_body,
        out_shape=jax.ShapeDtypeStruct((OUT_ROWS, D), jnp.float32),
        in_specs=[
            pl.BlockSpec(memory_space=pltpu.SMEM),
            pl.BlockSpec(memory_space=pltpu.VMEM),
            pl.BlockSpec(memory_space=pltpu.VMEM),
        ],
        out_specs=pl.BlockSpec(memory_space=pltpu.VMEM),
    )(alpha, parts, h)


def kernel(t, x, edge_index, alpha):
    h = x[: N * D].reshape(N, D)
    e = x[N * D:]
    parts = _sc_scatter(h, edge_index.reshape(-1), e)
    out = _tc_combine(jnp.reshape(alpha, (1, 1)), parts, h)
    return out.reshape(-1)
